# baseline (speedup 1.0000x reference)
import jax
import jax.numpy as jnp
from jax.experimental import pallas as pl
from jax.experimental.pallas import tpu as pltpu


def _body(pred_ref, true_ref, mask_ref, out_ref, acc_sum, acc_cnt):
    i = pl.program_id(0)
    n = pl.num_programs(0)
    m = mask_ref[...].astype(jnp.float32)
    d = jnp.abs(pred_ref[...] - true_ref[...])
    psum = jnp.sum(d * m)
    pcnt = jnp.sum(m)

    @pl.when(i == 0)
    def _():
        acc_sum[0] = 0.0
        acc_cnt[0] = 0.0

    acc_sum[0] += psum
    acc_cnt[0] += pcnt

    @pl.when(i == n - 1)
    def _():
        out_ref[0, 0] = jnp.sqrt(acc_sum[0] / acc_cnt[0])


def kernel(y_pred, y_true, mask):
    B, S, D = y_pred.shape
    rows = B * S
    yp = y_pred.reshape(rows, D)
    yt = y_true.reshape(rows, D)
    mk = mask.reshape(rows, D)

    block_rows = 256
    grid = rows // block_rows

    out = pl.pallas_call(
        _body,
        grid=(grid,),
        in_specs=[
            pl.BlockSpec((block_rows, D), lambda i: (i, 0)),
            pl.BlockSpec((block_rows, D), lambda i: (i, 0)),
            pl.BlockSpec((block_rows, D), lambda i: (i, 0)),
        ],
        out_specs=pl.BlockSpec(memory_space=pltpu.SMEM),
        out_shape=jax.ShapeDtypeStruct((1, 1), jnp.float32),
        scratch_shapes=[
            pltpu.SMEM((1,), jnp.float32),
            pltpu.SMEM((1,), jnp.float32),
        ],
    )(yp, yt, mk)
    return out[0, 0]


# speedup vs baseline: 1.0018x; 1.0018x over previous
import jax
import jax.numpy as jnp
from jax.experimental import pallas as pl
from jax.experimental.pallas import tpu as pltpu


def _body(pred_ref, true_ref, mask_ref, out_ref, acc_sum, acc_cnt):
    i = pl.program_id(0)
    n = pl.num_programs(0)
    m = mask_ref[...]
    d = jnp.abs(pred_ref[...] - true_ref[...])
    psum = jnp.sum(jnp.where(m, d, 0.0))
    pcnt = jnp.sum(jnp.where(m, 1.0, 0.0))

    @pl.when(i == 0)
    def _():
        acc_sum[0] = 0.0
        acc_cnt[0] = 0.0

    acc_sum[0] += psum
    acc_cnt[0] += pcnt

    @pl.when(i == n - 1)
    def _():
        out_ref[0, 0] = jnp.sqrt(acc_sum[0] / acc_cnt[0])


def kernel(y_pred, y_true, mask):
    B, S, D = y_pred.shape
    rows = B * S
    yp = y_pred.reshape(rows, D)
    yt = y_true.reshape(rows, D)
    mk = mask.reshape(rows, D)

    block_rows = 256
    grid = rows // block_rows

    out = pl.pallas_call(
        _body,
        grid=(grid,),
        in_specs=[
            pl.BlockSpec((block_rows, D), lambda i: (i, 0)),
            pl.BlockSpec((block_rows, D), lambda i: (i, 0)),
            pl.BlockSpec((block_rows, D), lambda i: (i, 0)),
        ],
        out_specs=pl.BlockSpec(memory_space=pltpu.SMEM),
        out_shape=jax.ShapeDtypeStruct((1, 1), jnp.float32),
        scratch_shapes=[
            pltpu.SMEM((1,), jnp.float32),
            pltpu.SMEM((1,), jnp.float32),
        ],
    )(yp, yt, mk)
    return out[0, 0]


# speedup vs baseline: 1.2063x; 1.2042x over previous
import jax
import jax.numpy as jnp
from jax.experimental import pallas as pl
from jax.experimental.pallas import tpu as pltpu


def _body(pred_ref, true_ref, mask_ref, out_ref, acc_sum, acc_cnt):
    i = pl.program_id(0)
    n = pl.num_programs(0)
    m = mask_ref[...]
    d = jnp.abs(pred_ref[...] - true_ref[...])
    psum = jnp.sum(jnp.where(m, d, 0.0))
    pcnt = jnp.sum(jnp.where(m, 1.0, 0.0))

    @pl.when(i == 0)
    def _():
        acc_sum[0] = 0.0
        acc_cnt[0] = 0.0

    acc_sum[0] += psum
    acc_cnt[0] += pcnt

    @pl.when(i == n - 1)
    def _():
        out_ref[0, 0] = jnp.sqrt(acc_sum[0] / acc_cnt[0])


def kernel(y_pred, y_true, mask):
    B, S, D = y_pred.shape
    rows = B * S
    yp = y_pred.reshape(rows, D)
    yt = y_true.reshape(rows, D)
    mk = mask.reshape(rows, D)

    block_rows = 512
    grid = rows // block_rows

    out = pl.pallas_call(
        _body,
        grid=(grid,),
        in_specs=[
            pl.BlockSpec((block_rows, D), lambda i: (i, 0)),
            pl.BlockSpec((block_rows, D), lambda i: (i, 0)),
            pl.BlockSpec((block_rows, D), lambda i: (i, 0)),
        ],
        out_specs=pl.BlockSpec(memory_space=pltpu.SMEM),
        out_shape=jax.ShapeDtypeStruct((1, 1), jnp.float32),
        scratch_shapes=[
            pltpu.SMEM((1,), jnp.float32),
            pltpu.SMEM((1,), jnp.float32),
        ],
    )(yp, yt, mk)
    return out[0, 0]


# speedup vs baseline: 1.2187x; 1.0102x over previous
import jax
import jax.numpy as jnp
from jax.experimental import pallas as pl
from jax.experimental.pallas import tpu as pltpu


def _body(pred_ref, true_ref, mask_ref, out_ref, acc_sum, acc_cnt):
    i = pl.program_id(0)
    n = pl.num_programs(0)
    m = mask_ref[...]
    d = jnp.abs(pred_ref[...] - true_ref[...])
    psum = jnp.sum(jnp.where(m, d, 0.0))
    pcnt = jnp.sum(jnp.where(m, 1.0, 0.0))

    @pl.when(i == 0)
    def _():
        acc_sum[0] = 0.0
        acc_cnt[0] = 0.0

    acc_sum[0] += psum
    acc_cnt[0] += pcnt

    @pl.when(i == n - 1)
    def _():
        out_ref[0, 0] = jnp.sqrt(acc_sum[0] / acc_cnt[0])


def kernel(y_pred, y_true, mask):
    B, S, D = y_pred.shape
    rows = B * S
    yp = y_pred.reshape(rows, D)
    yt = y_true.reshape(rows, D)
    mk = mask.reshape(rows, D)

    block_rows = 1024
    grid = rows // block_rows

    out = pl.pallas_call(
        _body,
        grid=(grid,),
        in_specs=[
            pl.BlockSpec((block_rows, D), lambda i: (i, 0)),
            pl.BlockSpec((block_rows, D), lambda i: (i, 0)),
            pl.BlockSpec((block_rows, D), lambda i: (i, 0)),
        ],
        out_specs=pl.BlockSpec(memory_space=pltpu.SMEM),
        out_shape=jax.ShapeDtypeStruct((1, 1), jnp.float32),
        scratch_shapes=[
            pltpu.SMEM((1,), jnp.float32),
            pltpu.SMEM((1,), jnp.float32),
        ],
    )(yp, yt, mk)
    return out[0, 0]
